# Initial kernel scaffold; baseline (speedup 1.0000x reference)
import jax
import jax.numpy as jnp
from jax.experimental import pallas as pl


def kernel(seq_len, row_embed, col_embed):
    raise NotImplementedError("write your pallas kernel here")


# speedup vs baseline: 1.2147x; 1.2147x over previous
import functools

import jax
import jax.numpy as jnp
from jax import lax
from jax.experimental import pallas as pl
from jax.experimental.pallas import tpu as pltpu
from jax.experimental.pallas import tpu_sc as plsc

GRID = 64
D_ROW = 1024
D_COL = 1024
D_MODEL = D_ROW + D_COL
SEQ = GRID * GRID

NC = 2
NS = 16
NW = NC * NS
ROWS_PER_W = SEQ // NW
CHUNK = 32
NCHUNK = ROWS_PER_W // CHUNK


@functools.partial(
    pl.kernel,
    mesh=plsc.VectorSubcoreMesh(core_axis_name="c", subcore_axis_name="s"),
    out_type=jax.ShapeDtypeStruct((SEQ, D_MODEL), jnp.float32),
    scratch_types=[
        pltpu.VMEM((CHUNK,), jnp.int32),
        pltpu.VMEM((CHUNK, D_ROW), jnp.float32),
        pltpu.VMEM((CHUNK, D_COL), jnp.float32),
        pltpu.SemaphoreType.DMA,
        pltpu.SemaphoreType.DMA,
    ],
)
def _pos_enc_sc(row_hbm, col_hbm, out_hbm, ridx_v, left_v, right_v, sem_l, sem_r):
    wid = lax.axis_index("s") * NC + lax.axis_index("c")
    for k in range(NCHUNK):
        base = pl.multiple_of(wid * ROWS_PER_W + k * CHUNK, CHUNK)
        r = base >> 6
        c0 = pl.multiple_of(base & (GRID - 1), CHUNK)
        for j in range(CHUNK // 16):
            ridx_v[pl.ds(j * 16, 16)] = jnp.zeros((16,), jnp.int32) + r
        cp_l = pltpu.async_copy(row_hbm.at[ridx_v], left_v, sem_l)
        cp_r = pltpu.async_copy(col_hbm.at[pl.ds(c0, CHUNK)], right_v, sem_r)
        cp_l.wait()
        pltpu.sync_copy(left_v, out_hbm.at[pl.ds(base, CHUNK), pl.ds(0, D_ROW)])
        cp_r.wait()
        pltpu.sync_copy(right_v, out_hbm.at[pl.ds(base, CHUNK), pl.ds(D_ROW, D_COL)])


def kernel(seq_len, row_embed, col_embed):
    del seq_len
    return _pos_enc_sc(row_embed, col_embed)


# speedup vs baseline: 2.0135x; 1.6576x over previous
import functools

import jax
import jax.numpy as jnp
from jax import lax
from jax.experimental import pallas as pl
from jax.experimental.pallas import tpu as pltpu
from jax.experimental.pallas import tpu_sc as plsc

GRID = 64
D_ROW = 1024
D_COL = 1024
D_MODEL = D_ROW + D_COL
SEQ = GRID * GRID

NC = 2
NS = 16
NW = NC * NS
HB = GRID // 2


@functools.partial(
    pl.kernel,
    mesh=plsc.VectorSubcoreMesh(core_axis_name="c", subcore_axis_name="s"),
    out_type=jax.ShapeDtypeStruct((SEQ, D_MODEL), jnp.float32),
    scratch_types=[
        pltpu.VMEM((1, D_ROW), jnp.float32),
        pltpu.VMEM((HB, D_ROW), jnp.float32),
        pltpu.VMEM((HB, D_COL), jnp.float32),
        pltpu.VMEM((HB, D_COL), jnp.float32),
        pltpu.SemaphoreType.DMA,
        pltpu.SemaphoreType.DMA,
        pltpu.SemaphoreType.DMA,
    ],
)
def _pos_enc_sc(row_hbm, col_hbm, out_hbm, rowbuf, left, col_a, col_b,
                sem_c, sem_lw, sem_rw):
    wid = lax.axis_index("s") * NC + lax.axis_index("c")

    cp_a = pltpu.async_copy(col_hbm.at[pl.ds(0, HB)], col_a, sem_c)
    cp_b = pltpu.async_copy(col_hbm.at[pl.ds(HB, HB)], col_b, sem_c)

    def replicate(j, _):
        off = pl.multiple_of(j * 16, 16)
        v = rowbuf[0, pl.ds(off, 16)]
        for i in range(HB):
            left[i, pl.ds(off, 16)] = v
        return 0

    right_writes = []
    for t in range(2):
        r = 2 * wid + t
        rbase = pl.multiple_of(r * GRID, GRID)
        pltpu.sync_copy(row_hbm.at[pl.ds(r, 1)], rowbuf)
        lax.fori_loop(0, D_ROW // 16, replicate, 0)
        wl0 = pltpu.async_copy(
            left, out_hbm.at[pl.ds(rbase, HB), pl.ds(0, D_ROW)], sem_lw)
        wl1 = pltpu.async_copy(
            left, out_hbm.at[pl.ds(rbase + HB, HB), pl.ds(0, D_ROW)], sem_lw)
        if t == 0:
            cp_a.wait()
            cp_b.wait()
        right_writes.append(pltpu.async_copy(
            col_a, out_hbm.at[pl.ds(rbase, HB), pl.ds(D_ROW, D_COL)], sem_rw))
        right_writes.append(pltpu.async_copy(
            col_b, out_hbm.at[pl.ds(rbase + HB, HB), pl.ds(D_ROW, D_COL)],
            sem_rw))
        wl0.wait()
        wl1.wait()
    for w in right_writes:
        w.wait()


def kernel(seq_len, row_embed, col_embed):
    del seq_len
    return _pos_enc_sc(row_embed, col_embed)
